# baseline (speedup 1.0000x reference)
import jax
import jax.numpy as jnp
from jax import lax
from jax.experimental import pallas as pl
from jax.experimental.pallas import tpu as pltpu
from jax.experimental.pallas import tpu_sc as plsc

_INF = 100000000.0
_BACKGROUND = 8
_RADIUS = 1.5

_N = 21824
_NW = 32
_PER_W = 688
_NVEC = _PER_W // 16
_G = 100
_GV = 7

_f32 = jnp.float32
_i32 = jnp.int32


def _sqrt16(x):
    i = plsc.bitcast(x, _i32)
    y = plsc.bitcast(jnp.int32(0x5F3759DF) - (i >> 1), _f32)
    for _ in range(3):
        y = y * (1.5 - 0.5 * x * y * y)
    return x * y


def _body(pts_h, gt_h, glab_h, rr_h, st_h,
          lab_o, bbox_o, ctr_o,
          pts_v, rr_v, st_v, gt_v, glab_v,
          cx_v, cy_v, ar_v, hm_v, gidx_v,
          lab_v, bbox_v, ctr_v, sem):
    wid = lax.axis_index("s") * 2 + lax.axis_index("c")
    base = jnp.minimum(wid * _PER_W, _N - _PER_W)

    cp = [
        pltpu.async_copy(pts_h.at[pl.ds(base, _PER_W), :], pts_v, sem),
        pltpu.async_copy(rr_h.at[pl.ds(base, _PER_W), :], rr_v, sem),
        pltpu.async_copy(st_h.at[pl.ds(base, _PER_W)], st_v, sem),
        pltpu.async_copy(gt_h, gt_v, sem),
        pltpu.async_copy(glab_h, glab_v, sem),
    ]
    for c in cp:
        c.wait()

    iota16 = jnp.arange(16, dtype=_i32)
    c0 = jnp.zeros((16,), _i32)
    c1 = jnp.full((16,), 1, _i32)
    c2 = jnp.full((16,), 2, _i32)
    c3 = jnp.full((16,), 3, _i32)

    for j in range(_GV):
        sl = pl.ds(j * 16, 16)
        gvec = iota16 + (j * 16)
        gvm = jnp.minimum(gvec, _G - 1)
        x1 = plsc.load_gather(gt_v, [gvm, c0])
        y1 = plsc.load_gather(gt_v, [gvm, c1])
        x2 = plsc.load_gather(gt_v, [gvm, c2])
        y2 = plsc.load_gather(gt_v, [gvm, c3])
        cx_v[sl] = (x1 + x2) * 0.5
        cy_v[sl] = (y1 + y2) * 0.5
        ar_v[sl] = (x2 - x1) * (y2 - y1)
        hm_v[sl] = jnp.maximum(x2 - x1, y2 - y1)

    def point_vec(i, _):
        off = i * 16
        sl = pl.ds(off, 16)
        ridx = iota16 + off
        xs = plsc.load_gather(pts_v, [ridx, c0])
        ys = plsc.load_gather(pts_v, [ridx, c1])
        rlo = plsc.load_gather(rr_v, [ridx, c0])
        rhi = plsc.load_gather(rr_v, [ridx, c1])
        st = st_v[sl]
        rad = st * _RADIUS

        pxmn = jnp.min(xs)
        pxmx = jnp.max(xs)
        pymn = jnp.min(ys)
        pymx = jnp.max(ys)
        radv = jnp.max(rad)
        rlomn = jnp.min(rlo)
        rhimx2 = jnp.max(rhi) * 2.0
        tot = jnp.int32(0)
        for j in range(_GV):
            gsl = pl.ds(j * 16, 16)
            gvec = iota16 + (j * 16)
            gvm = jnp.minimum(gvec, _G - 1)
            x1 = plsc.load_gather(gt_v, [gvm, c0])
            y1 = plsc.load_gather(gt_v, [gvm, c1])
            x2 = plsc.load_gather(gt_v, [gvm, c2])
            y2 = plsc.load_gather(gt_v, [gvm, c3])
            cx = cx_v[gsl]
            cy = cy_v[gsl]
            hm = hm_v[gsl]
            cgx1 = jnp.maximum(cx - radv, x1)
            cgx2 = jnp.minimum(cx + radv, x2)
            cgy1 = jnp.maximum(cy - radv, y1)
            cgy2 = jnp.minimum(cy + radv, y2)
            keep = (cgx1 < pxmx) & (cgx2 > pxmn) & (cgy1 < pymx) & (cgy2 > pymn)
            keep &= (hm >= rlomn) & (hm <= rhimx2)
            keep &= gvec < _G
            plsc.store_compressed(gidx_v.at[pl.ds(tot, 16)], gvec, mask=keep)
            tot = tot + jnp.sum(keep.astype(_i32))

        def per_gt(k, carry):
            min_area, min_idx = carry
            ki = jnp.full((16,), k, _i32)
            gi = plsc.load_gather(gidx_v, [ki])
            x1 = plsc.load_gather(gt_v, [gi, c0])
            y1 = plsc.load_gather(gt_v, [gi, c1])
            x2 = plsc.load_gather(gt_v, [gi, c2])
            y2 = plsc.load_gather(gt_v, [gi, c3])
            cx = plsc.load_gather(cx_v, [gi])
            cy = plsc.load_gather(cy_v, [gi])
            ar = plsc.load_gather(ar_v, [gi])
            l = xs - x1
            t = ys - y1
            r = x2 - xs
            b = y2 - ys
            maxreg = jnp.maximum(jnp.maximum(l, t), jnp.maximum(r, b))
            in_rr = (maxreg >= rlo) & (maxreg <= rhi)
            cgx1 = jnp.maximum(cx - rad, x1)
            cgy1 = jnp.maximum(cy - rad, y1)
            cgx2 = jnp.minimum(cx + rad, x2)
            cgy2 = jnp.minimum(cy + rad, y2)
            m = jnp.minimum(jnp.minimum(xs - cgx1, ys - cgy1),
                            jnp.minimum(cgx2 - xs, cgy2 - ys))
            cond = (m > 0) & in_rr
            a_m = jnp.where(cond, ar, _INF)
            better = a_m < min_area
            return jnp.minimum(min_area, a_m), jnp.where(better, gi, min_idx)

        init = (jnp.full((16,), _INF, _f32), jnp.zeros((16,), _i32))
        min_area, min_idx = lax.fori_loop(0, tot, per_gt, init)

        wx1 = plsc.load_gather(gt_v, [min_idx, c0])
        wy1 = plsc.load_gather(gt_v, [min_idx, c1])
        wx2 = plsc.load_gather(gt_v, [min_idx, c2])
        wy2 = plsc.load_gather(gt_v, [min_idx, c3])
        wl = plsc.load_gather(glab_v, [min_idx])
        l = xs - wx1
        t = ys - wy1
        r = wx2 - xs
        b = wy2 - ys
        lr_min = jnp.minimum(l, r)
        lr_max = jnp.maximum(l, r)
        tb_min = jnp.minimum(t, b)
        tb_max = jnp.maximum(t, b)
        ratio = (lr_min / jnp.maximum(lr_max, 1e-6)) * (tb_min / jnp.maximum(tb_max, 1e-6))
        ctr = _sqrt16(jnp.maximum(ratio, 1e-12))
        is_bg = min_area >= _INF
        lab_v[sl] = jnp.where(is_bg, _BACKGROUND, wl)
        ctr_v[sl] = ctr
        plsc.store_scatter(bbox_v, [ridx, c0], l / st)
        plsc.store_scatter(bbox_v, [ridx, c1], t / st)
        plsc.store_scatter(bbox_v, [ridx, c2], r / st)
        plsc.store_scatter(bbox_v, [ridx, c3], b / st)
        return _

    lax.fori_loop(0, _NVEC, point_vec, 0)

    out = [
        pltpu.async_copy(lab_v, lab_o.at[pl.ds(base, _PER_W)], sem),
        pltpu.async_copy(bbox_v, bbox_o.at[pl.ds(base, _PER_W), :], sem),
        pltpu.async_copy(ctr_v, ctr_o.at[pl.ds(base, _PER_W)], sem),
    ]
    for c in out:
        c.wait()


_sc_call = pl.kernel(
    _body,
    out_type=(
        jax.ShapeDtypeStruct((_N,), _i32),
        jax.ShapeDtypeStruct((_N, 4), _f32),
        jax.ShapeDtypeStruct((_N,), _f32),
    ),
    mesh=plsc.VectorSubcoreMesh(
        core_axis_name="c", subcore_axis_name="s", num_cores=2, num_subcores=16
    ),
    compiler_params=pltpu.CompilerParams(
        needs_layout_passes=False, use_tc_tiling_on_sc=False),
    scratch_types=[
        pltpu.VMEM((_PER_W, 2), _f32),
        pltpu.VMEM((_PER_W, 2), _f32),
        pltpu.VMEM((_PER_W,), _f32),
        pltpu.VMEM((_G, 4), _f32),
        pltpu.VMEM((_G,), _i32),
        pltpu.VMEM((_GV * 16,), _f32),
        pltpu.VMEM((_GV * 16,), _f32),
        pltpu.VMEM((_GV * 16,), _f32),
        pltpu.VMEM((_GV * 16,), _f32),
        pltpu.VMEM((128,), _i32),
        pltpu.VMEM((_PER_W,), _i32),
        pltpu.VMEM((_PER_W, 4), _f32),
        pltpu.VMEM((_PER_W,), _f32),
        pltpu.SemaphoreType.DMA,
    ],
)


def kernel(points, gt_bboxes, gt_labels, regress_ranges, strides_per_point):
    return _sc_call(points, gt_bboxes, gt_labels.astype(_i32),
                    regress_ranges, strides_per_point)


# speedup vs baseline: 1.1507x; 1.1507x over previous
import jax
import jax.numpy as jnp
from jax import lax
from jax.experimental import pallas as pl
from jax.experimental.pallas import tpu as pltpu
from jax.experimental.pallas import tpu_sc as plsc

_INF = 100000000.0
_BACKGROUND = 8
_RADIUS = 1.5

_N = 21824
_NW = 32
_PER_W = 688
_NVEC = _PER_W // 16
_G = 100
_GV = 7

_f32 = jnp.float32
_i32 = jnp.int32


def _sqrt16(x):
    i = plsc.bitcast(x, _i32)
    y = plsc.bitcast(jnp.int32(0x5F3759DF) - (i >> 1), _f32)
    for _ in range(3):
        y = y * (1.5 - 0.5 * x * y * y)
    return x * y


def _body(pts_h, gt_h, glab_h, rr_h, st_h,
          lab_o, bbox_o, ctr_o,
          pts_v, rr_v, st_v, gt_v, glab_v,
          cx_v, cy_v, ar_v, hm_v, gidx_v,
          lab_v, bbox_v, ctr_v, sem):
    wid = lax.axis_index("s") * 2 + lax.axis_index("c")
    base = jnp.minimum(wid * _PER_W, _N - _PER_W)

    cp = [
        pltpu.async_copy(pts_h.at[pl.ds(base * 2, _PER_W * 2)], pts_v, sem),
        pltpu.async_copy(rr_h.at[pl.ds(base * 2, _PER_W * 2)], rr_v, sem),
        pltpu.async_copy(st_h.at[pl.ds(base, _PER_W)], st_v, sem),
        pltpu.async_copy(gt_h, gt_v, sem),
        pltpu.async_copy(glab_h, glab_v, sem),
    ]
    for c in cp:
        c.wait()

    iota16 = jnp.arange(16, dtype=_i32)

    for j in range(_GV):
        sl = pl.ds(j * 16, 16)
        gvec = iota16 + (j * 16)
        g4 = jnp.minimum(gvec, _G - 1) * 4
        x1 = plsc.load_gather(gt_v, [g4])
        y1 = plsc.load_gather(gt_v, [g4 + 1])
        x2 = plsc.load_gather(gt_v, [g4 + 2])
        y2 = plsc.load_gather(gt_v, [g4 + 3])
        cx_v[sl] = (x1 + x2) * 0.5
        cy_v[sl] = (y1 + y2) * 0.5
        ar_v[sl] = (x2 - x1) * (y2 - y1)
        hm_v[sl] = jnp.maximum(x2 - x1, y2 - y1)

    def point_vec(i, _):
        off = i * 16
        sl = pl.ds(off, 16)
        r2 = (iota16 + off) * 2
        xs = plsc.load_gather(pts_v, [r2])
        ys = plsc.load_gather(pts_v, [r2 + 1])
        rlo = plsc.load_gather(rr_v, [r2])
        rhi = plsc.load_gather(rr_v, [r2 + 1])
        st = st_v[sl]
        rad = st * _RADIUS

        pxmn = jnp.min(xs)
        pxmx = jnp.max(xs)
        pymn = jnp.min(ys)
        pymx = jnp.max(ys)
        radv = jnp.max(rad)
        rlomn = jnp.min(rlo)
        rhimx2 = jnp.max(rhi) * 2.0
        tot = jnp.int32(0)
        for j in range(_GV):
            gsl = pl.ds(j * 16, 16)
            gvec = iota16 + (j * 16)
            g4 = jnp.minimum(gvec, _G - 1) * 4
            x1 = plsc.load_gather(gt_v, [g4])
            y1 = plsc.load_gather(gt_v, [g4 + 1])
            x2 = plsc.load_gather(gt_v, [g4 + 2])
            y2 = plsc.load_gather(gt_v, [g4 + 3])
            cx = cx_v[gsl]
            cy = cy_v[gsl]
            hm = hm_v[gsl]
            cgx1 = jnp.maximum(cx - radv, x1)
            cgx2 = jnp.minimum(cx + radv, x2)
            cgy1 = jnp.maximum(cy - radv, y1)
            cgy2 = jnp.minimum(cy + radv, y2)
            keep = (cgx1 < pxmx) & (cgx2 > pxmn) & (cgy1 < pymx) & (cgy2 > pymn)
            keep &= (hm >= rlomn) & (hm <= rhimx2)
            keep &= gvec < _G
            plsc.store_compressed(gidx_v.at[pl.ds(tot, 16)], gvec, mask=keep)
            tot = tot + jnp.sum(keep.astype(_i32))

        def per_gt(k, carry):
            min_area, min_idx = carry
            ki = jnp.full((16,), k, _i32)
            gi = plsc.load_gather(gidx_v, [ki])
            g4 = gi * 4
            x1 = plsc.load_gather(gt_v, [g4])
            y1 = plsc.load_gather(gt_v, [g4 + 1])
            x2 = plsc.load_gather(gt_v, [g4 + 2])
            y2 = plsc.load_gather(gt_v, [g4 + 3])
            cx = plsc.load_gather(cx_v, [gi])
            cy = plsc.load_gather(cy_v, [gi])
            ar = plsc.load_gather(ar_v, [gi])
            l = xs - x1
            t = ys - y1
            r = x2 - xs
            b = y2 - ys
            maxreg = jnp.maximum(jnp.maximum(l, t), jnp.maximum(r, b))
            in_rr = (maxreg >= rlo) & (maxreg <= rhi)
            cgx1 = jnp.maximum(cx - rad, x1)
            cgy1 = jnp.maximum(cy - rad, y1)
            cgx2 = jnp.minimum(cx + rad, x2)
            cgy2 = jnp.minimum(cy + rad, y2)
            m = jnp.minimum(jnp.minimum(xs - cgx1, ys - cgy1),
                            jnp.minimum(cgx2 - xs, cgy2 - ys))
            cond = (m > 0) & in_rr
            a_m = jnp.where(cond, ar, _INF)
            better = a_m < min_area
            return jnp.minimum(min_area, a_m), jnp.where(better, gi, min_idx)

        init = (jnp.full((16,), _INF, _f32), jnp.zeros((16,), _i32))
        min_area, min_idx = lax.fori_loop(0, tot, per_gt, init)

        w4 = min_idx * 4
        wx1 = plsc.load_gather(gt_v, [w4])
        wy1 = plsc.load_gather(gt_v, [w4 + 1])
        wx2 = plsc.load_gather(gt_v, [w4 + 2])
        wy2 = plsc.load_gather(gt_v, [w4 + 3])
        wl = plsc.load_gather(glab_v, [min_idx])
        l = xs - wx1
        t = ys - wy1
        r = wx2 - xs
        b = wy2 - ys
        lr_min = jnp.minimum(l, r)
        lr_max = jnp.maximum(l, r)
        tb_min = jnp.minimum(t, b)
        tb_max = jnp.maximum(t, b)
        ratio = (lr_min / jnp.maximum(lr_max, 1e-6)) * (tb_min / jnp.maximum(tb_max, 1e-6))
        ctr = _sqrt16(jnp.maximum(ratio, 1e-12))
        is_bg = min_area >= _INF
        lab_v[sl] = jnp.where(is_bg, _BACKGROUND, wl)
        ctr_v[sl] = ctr
        b4 = r2 * 2
        plsc.store_scatter(bbox_v, [b4], l / st)
        plsc.store_scatter(bbox_v, [b4 + 1], t / st)
        plsc.store_scatter(bbox_v, [b4 + 2], r / st)
        plsc.store_scatter(bbox_v, [b4 + 3], b / st)
        return _

    lax.fori_loop(0, _NVEC, point_vec, 0)

    out = [
        pltpu.async_copy(lab_v, lab_o.at[pl.ds(base, _PER_W)], sem),
        pltpu.async_copy(bbox_v, bbox_o.at[pl.ds(base * 4, _PER_W * 4)], sem),
        pltpu.async_copy(ctr_v, ctr_o.at[pl.ds(base, _PER_W)], sem),
    ]
    for c in out:
        c.wait()


_sc_call = pl.kernel(
    _body,
    out_type=(
        jax.ShapeDtypeStruct((_N,), _i32),
        jax.ShapeDtypeStruct((_N * 4,), _f32),
        jax.ShapeDtypeStruct((_N,), _f32),
    ),
    mesh=plsc.VectorSubcoreMesh(
        core_axis_name="c", subcore_axis_name="s", num_cores=2, num_subcores=16
    ),
    compiler_params=pltpu.CompilerParams(
        needs_layout_passes=False, use_tc_tiling_on_sc=False),
    scratch_types=[
        pltpu.VMEM((_PER_W * 2,), _f32),
        pltpu.VMEM((_PER_W * 2,), _f32),
        pltpu.VMEM((_PER_W,), _f32),
        pltpu.VMEM((_G * 4,), _f32),
        pltpu.VMEM((_G,), _i32),
        pltpu.VMEM((_GV * 16,), _f32),
        pltpu.VMEM((_GV * 16,), _f32),
        pltpu.VMEM((_GV * 16,), _f32),
        pltpu.VMEM((_GV * 16,), _f32),
        pltpu.VMEM((128,), _i32),
        pltpu.VMEM((_PER_W,), _i32),
        pltpu.VMEM((_PER_W * 4,), _f32),
        pltpu.VMEM((_PER_W,), _f32),
        pltpu.SemaphoreType.DMA,
    ],
)


def kernel(points, gt_bboxes, gt_labels, regress_ranges, strides_per_point):
    lab, bbox_flat, ctr = _sc_call(
        points.reshape(-1), gt_bboxes.reshape(-1), gt_labels.astype(_i32),
        regress_ranges.reshape(-1), strides_per_point)
    return lab, bbox_flat.reshape(_N, 4), ctr


# speedup vs baseline: 2.5746x; 2.2374x over previous
import jax
import jax.numpy as jnp
from jax import lax
from jax.experimental import pallas as pl
from jax.experimental.pallas import tpu as pltpu
from jax.experimental.pallas import tpu_sc as plsc

_INF = 100000000.0
_BACKGROUND = 8
_RADIUS = 1.5

_N = 21824
_NW = 32
_PER_W = 688
_NVEC = _PER_W // 16
_G = 100
_GV = 7

_f32 = jnp.float32
_i32 = jnp.int32


def _sqrt16(x):
    i = plsc.bitcast(x, _i32)
    y = plsc.bitcast(jnp.int32(0x5F3759DF) - (i >> 1), _f32)
    for _ in range(3):
        y = y * (1.5 - 0.5 * x * y * y)
    return x * y


def _body(xs_h, ys_h, st_h, rlo_h, rhi_h, gx1_h, gy1_h, gx2_h, gy2_h, glab_h,
          lab_o, bl_o, bt_o, br_o, bb_o, ctr_o,
          xs_v, ys_v, st_v, rlo_v, rhi_v,
          gx1_v, gy1_v, gx2_v, gy2_v, glab_v,
          cx_v, cy_v, ar_v, hm_v, gidx_v,
          lab_v, bl_v, bt_v, br_v, bb_v, ctr_v, sem):
    wid = lax.axis_index("s") * 2 + lax.axis_index("c")
    base = jnp.minimum(wid * _PER_W, _N - _PER_W)

    cp = [
        pltpu.async_copy(xs_h.at[pl.ds(base, _PER_W)], xs_v, sem),
        pltpu.async_copy(ys_h.at[pl.ds(base, _PER_W)], ys_v, sem),
        pltpu.async_copy(st_h.at[pl.ds(base, _PER_W)], st_v, sem),
        pltpu.async_copy(rlo_h.at[pl.ds(base, _PER_W)], rlo_v, sem),
        pltpu.async_copy(rhi_h.at[pl.ds(base, _PER_W)], rhi_v, sem),
        pltpu.async_copy(gx1_h, gx1_v.at[pl.ds(0, _G)], sem),
        pltpu.async_copy(gy1_h, gy1_v.at[pl.ds(0, _G)], sem),
        pltpu.async_copy(gx2_h, gx2_v.at[pl.ds(0, _G)], sem),
        pltpu.async_copy(gy2_h, gy2_v.at[pl.ds(0, _G)], sem),
        pltpu.async_copy(glab_h, glab_v.at[pl.ds(0, _G)], sem),
    ]
    for c in cp:
        c.wait()

    iota16 = jnp.arange(16, dtype=_i32)

    for j in range(_GV):
        sl = pl.ds(j * 16, 16)
        x1 = gx1_v[sl]
        y1 = gy1_v[sl]
        x2 = gx2_v[sl]
        y2 = gy2_v[sl]
        cx_v[sl] = (x1 + x2) * 0.5
        cy_v[sl] = (y1 + y2) * 0.5
        ar_v[sl] = (x2 - x1) * (y2 - y1)
        hm_v[sl] = jnp.maximum(x2 - x1, y2 - y1)

    def point_vec(i, _):
        off = i * 16
        sl = pl.ds(off, 16)
        xs = xs_v[sl]
        ys = ys_v[sl]
        rlo = rlo_v[sl]
        rhi = rhi_v[sl]
        st = st_v[sl]
        rad = st * _RADIUS

        pxmn = jnp.min(xs)
        pxmx = jnp.max(xs)
        pymn = jnp.min(ys)
        pymx = jnp.max(ys)
        radv = jnp.max(rad)
        rlomn = jnp.min(rlo)
        rhimx2 = jnp.max(rhi) * 2.0
        tot = jnp.int32(0)
        for j in range(_GV):
            gsl = pl.ds(j * 16, 16)
            gvec = iota16 + (j * 16)
            x1 = gx1_v[gsl]
            y1 = gy1_v[gsl]
            x2 = gx2_v[gsl]
            y2 = gy2_v[gsl]
            cx = cx_v[gsl]
            cy = cy_v[gsl]
            hm = hm_v[gsl]
            cgx1 = jnp.maximum(cx - radv, x1)
            cgx2 = jnp.minimum(cx + radv, x2)
            cgy1 = jnp.maximum(cy - radv, y1)
            cgy2 = jnp.minimum(cy + radv, y2)
            keep = (cgx1 < pxmx) & (cgx2 > pxmn) & (cgy1 < pymx) & (cgy2 > pymn)
            keep &= (hm >= rlomn) & (hm <= rhimx2)
            keep &= gvec < _G
            plsc.store_compressed(gidx_v.at[pl.ds(tot, 16)], gvec, mask=keep)
            tot = tot + jnp.sum(keep.astype(_i32))

        def per_gt(k, carry):
            min_area, min_idx = carry
            ki = jnp.full((16,), k, _i32)
            gi = plsc.load_gather(gidx_v, [ki])
            x1 = plsc.load_gather(gx1_v, [gi])
            y1 = plsc.load_gather(gy1_v, [gi])
            x2 = plsc.load_gather(gx2_v, [gi])
            y2 = plsc.load_gather(gy2_v, [gi])
            cx = plsc.load_gather(cx_v, [gi])
            cy = plsc.load_gather(cy_v, [gi])
            ar = plsc.load_gather(ar_v, [gi])
            l = xs - x1
            t = ys - y1
            r = x2 - xs
            b = y2 - ys
            maxreg = jnp.maximum(jnp.maximum(l, t), jnp.maximum(r, b))
            in_rr = (maxreg >= rlo) & (maxreg <= rhi)
            cgx1 = jnp.maximum(cx - rad, x1)
            cgy1 = jnp.maximum(cy - rad, y1)
            cgx2 = jnp.minimum(cx + rad, x2)
            cgy2 = jnp.minimum(cy + rad, y2)
            m = jnp.minimum(jnp.minimum(xs - cgx1, ys - cgy1),
                            jnp.minimum(cgx2 - xs, cgy2 - ys))
            cond = (m > 0) & in_rr
            a_m = jnp.where(cond, ar, _INF)
            better = a_m < min_area
            return jnp.minimum(min_area, a_m), jnp.where(better, gi, min_idx)

        init = (jnp.full((16,), _INF, _f32), jnp.zeros((16,), _i32))
        min_area, min_idx = lax.fori_loop(0, tot, per_gt, init)

        wx1 = plsc.load_gather(gx1_v, [min_idx])
        wy1 = plsc.load_gather(gy1_v, [min_idx])
        wx2 = plsc.load_gather(gx2_v, [min_idx])
        wy2 = plsc.load_gather(gy2_v, [min_idx])
        wl = plsc.load_gather(glab_v, [min_idx])
        l = xs - wx1
        t = ys - wy1
        r = wx2 - xs
        b = wy2 - ys
        lr_min = jnp.minimum(l, r)
        lr_max = jnp.maximum(l, r)
        tb_min = jnp.minimum(t, b)
        tb_max = jnp.maximum(t, b)
        ratio = (lr_min / jnp.maximum(lr_max, 1e-6)) * (tb_min / jnp.maximum(tb_max, 1e-6))
        ctr = _sqrt16(jnp.maximum(ratio, 1e-12))
        is_bg = min_area >= _INF
        lab_v[sl] = jnp.where(is_bg, _BACKGROUND, wl)
        ctr_v[sl] = ctr
        bl_v[sl] = l / st
        bt_v[sl] = t / st
        br_v[sl] = r / st
        bb_v[sl] = b / st
        return _

    lax.fori_loop(0, _NVEC, point_vec, 0)

    out = [
        pltpu.async_copy(lab_v, lab_o.at[pl.ds(base, _PER_W)], sem),
        pltpu.async_copy(bl_v, bl_o.at[pl.ds(base, _PER_W)], sem),
        pltpu.async_copy(bt_v, bt_o.at[pl.ds(base, _PER_W)], sem),
        pltpu.async_copy(br_v, br_o.at[pl.ds(base, _PER_W)], sem),
        pltpu.async_copy(bb_v, bb_o.at[pl.ds(base, _PER_W)], sem),
        pltpu.async_copy(ctr_v, ctr_o.at[pl.ds(base, _PER_W)], sem),
    ]
    for c in out:
        c.wait()


_sc_call = pl.kernel(
    _body,
    out_type=tuple(
        jax.ShapeDtypeStruct((_N,), dt)
        for dt in (_i32, _f32, _f32, _f32, _f32, _f32)
    ),
    mesh=plsc.VectorSubcoreMesh(
        core_axis_name="c", subcore_axis_name="s", num_cores=2, num_subcores=16
    ),
    compiler_params=pltpu.CompilerParams(
        needs_layout_passes=False, use_tc_tiling_on_sc=False),
    scratch_types=[
        pltpu.VMEM((_PER_W,), _f32),
        pltpu.VMEM((_PER_W,), _f32),
        pltpu.VMEM((_PER_W,), _f32),
        pltpu.VMEM((_PER_W,), _f32),
        pltpu.VMEM((_PER_W,), _f32),
        pltpu.VMEM((_GV * 16,), _f32),
        pltpu.VMEM((_GV * 16,), _f32),
        pltpu.VMEM((_GV * 16,), _f32),
        pltpu.VMEM((_GV * 16,), _f32),
        pltpu.VMEM((_GV * 16,), _i32),
        pltpu.VMEM((_GV * 16,), _f32),
        pltpu.VMEM((_GV * 16,), _f32),
        pltpu.VMEM((_GV * 16,), _f32),
        pltpu.VMEM((_GV * 16,), _f32),
        pltpu.VMEM((128,), _i32),
        pltpu.VMEM((_PER_W,), _i32),
        pltpu.VMEM((_PER_W,), _f32),
        pltpu.VMEM((_PER_W,), _f32),
        pltpu.VMEM((_PER_W,), _f32),
        pltpu.VMEM((_PER_W,), _f32),
        pltpu.VMEM((_PER_W,), _f32),
        pltpu.SemaphoreType.DMA,
    ],
)


def kernel(points, gt_bboxes, gt_labels, regress_ranges, strides_per_point):
    lab, bl, bt, br, bb, ctr = _sc_call(
        points[:, 0], points[:, 1], strides_per_point,
        regress_ranges[:, 0], regress_ranges[:, 1],
        gt_bboxes[:, 0], gt_bboxes[:, 1], gt_bboxes[:, 2], gt_bboxes[:, 3],
        gt_labels.astype(_i32))
    return lab, jnp.stack([bl, bt, br, bb], axis=-1), ctr
